# baseline (speedup 1.0000x reference)
import functools

import jax
import jax.numpy as jnp
from jax import lax
from jax.experimental import pallas as pl
from jax.experimental.pallas import tpu as pltpu
from jax.experimental.pallas import tpu_sc as plsc

B = 4096
F = 26
E = 64
N = B * F
NC, NS = 2, 16
NW = NC * NS
PER_W = N // NW
CH = 128
NCH = PER_W // CH


W1W = 16


def _sc_gather(v_table, w1_table, idx_flat):
    mesh = plsc.VectorSubcoreMesh(core_axis_name="c", subcore_axis_name="s")

    @functools.partial(
        pl.kernel,
        mesh=mesh,
        compiler_params=pltpu.CompilerParams(use_tc_tiling_on_sc=False),
        out_type=[
            jax.ShapeDtypeStruct((N, E), jnp.float32),
            jax.ShapeDtypeStruct((N, W1W), jnp.float32),
        ],
        scratch_types=[
            pltpu.VMEM((PER_W,), jnp.int32),
            pltpu.VMEM((CH, E), jnp.float32),
            pltpu.VMEM((CH, W1W), jnp.float32),
            pltpu.SemaphoreType.DMA,
            pltpu.SemaphoreType.DMA,
        ],
    )
    def k(vt_hbm, wt_hbm, idx_hbm, vout_hbm, wout_hbm, idx_v, vbuf, wbuf, sem0, sem1):
        wid = lax.axis_index("s") * NC + lax.axis_index("c")
        base = wid * PER_W
        pltpu.sync_copy(idx_hbm.at[pl.ds(base, PER_W)], idx_v)

        @pl.loop(0, NCH)
        def _(ci):
            off = ci * CH
            c1 = pltpu.async_copy(vt_hbm.at[idx_v.at[pl.ds(off, CH)]], vbuf, sem0)
            c2 = pltpu.async_copy(wt_hbm.at[idx_v.at[pl.ds(off, CH)]], wbuf, sem1)
            c1.wait()
            c2.wait()
            pltpu.sync_copy(vbuf, vout_hbm.at[pl.ds(base + off, CH)])
            pltpu.sync_copy(wbuf, wout_hbm.at[pl.ds(base + off, CH)])

    return k(v_table, w1_table, idx_flat)


BS = 512


def _tc_body(rows_ref, w1_ref, w0_ref, wl1_ref, bl1_ref, wl2_ref, bl2_ref,
             wl3_ref, bl3_ref, out_ref):
    rows = rows_ref[...]
    semb = rows[:, 0:E]
    qq = semb * semb
    for f in range(1, F):
        r = rows[:, f * E:(f + 1) * E]
        semb = semb + r
        qq = qq + r * r
    pp = jnp.sum(semb * semb + qq, axis=1, keepdims=True) * (1.0 / E)
    w1 = w1_ref[...]
    lin = w1[:, 0:1]
    for f in range(1, F):
        lin = lin + w1[:, f * W1W:f * W1W + 1]
    lin = lin + w0_ref[...]
    memb = semb * (1.0 / F)
    h = jnp.dot(memb, wl1_ref[...], preferred_element_type=jnp.float32) + bl1_ref[...]
    h = jnp.where(h >= 0, h, 0.01 * h)
    h = jnp.dot(h, wl2_ref[...], preferred_element_type=jnp.float32) + bl2_ref[...]
    h = jnp.where(h >= 0, h, 0.01 * h)
    deep = jnp.dot(h, wl3_ref[...], preferred_element_type=jnp.float32) + bl3_ref[...]
    out_ref[...] = lin + 0.5 * pp + deep


def _tc_forward(rows2, w1r, w0, wl1, bl1, wl2, bl2, wl3, bl3):
    grid = (B // BS,)
    full = lambda shape: pl.BlockSpec(shape, lambda i: (0, 0))
    return pl.pallas_call(
        _tc_body,
        grid=grid,
        in_specs=[
            pl.BlockSpec((BS, F * E), lambda i: (i, 0)),
            pl.BlockSpec((BS, F * W1W), lambda i: (i, 0)),
            full((1, 1)),
            full((E, 256)),
            full((1, 256)),
            full((256, 128)),
            full((1, 128)),
            full((128, 1)),
            full((1, 1)),
        ],
        out_specs=pl.BlockSpec((BS, 1), lambda i: (i, 0)),
        out_shape=jax.ShapeDtypeStruct((B, 1), jnp.float32),
    )(rows2, w1r, w0, wl1, bl1, wl2, bl2, wl3, bl3)


def kernel(x, W0, W1_table, V_table, W_l1, b_l1, W_l2, b_l2, W_l3, b_l3):
    idx_flat = x.reshape(-1).astype(jnp.int32)
    w1_pad = jnp.pad(W1_table, ((0, 0), (0, W1W - 1)))
    vrows, w1rows = _sc_gather(V_table, w1_pad, idx_flat)
    rows2 = vrows.reshape(B, F * E)
    w1r = w1rows.reshape(B, F * W1W)
    return _tc_forward(
        rows2, w1r,
        W0.reshape(1, 1),
        W_l1, b_l1.reshape(1, 256),
        W_l2, b_l2.reshape(1, 128),
        W_l3, b_l3.reshape(1, 1),
    )


# speedup vs baseline: 2.2194x; 2.2194x over previous
import functools

import jax
import jax.numpy as jnp
from jax import lax
from jax.experimental import pallas as pl
from jax.experimental.pallas import tpu as pltpu
from jax.experimental.pallas import tpu_sc as plsc

B = 4096
F = 26
E = 64
N = B * F
VOCAB = 100000
NC, NS = 2, 16
NW = NC * NS
PER_W = N // NW
SPW = B // NW
S_PER_CH = 4
CH = S_PER_CH * F
NCH = PER_W // CH
W1W = 16
WCH = 128
NWCH = PER_W // WCH


def _sc_fused(v_table, w1r16, idx_flat):
    mesh = plsc.VectorSubcoreMesh(core_axis_name="c", subcore_axis_name="s")

    @functools.partial(
        pl.kernel,
        mesh=mesh,
        compiler_params=pltpu.CompilerParams(use_tc_tiling_on_sc=False,
                                             needs_layout_passes=False),
        out_type=[
            jax.ShapeDtypeStruct((B, E), jnp.float32),
            jax.ShapeDtypeStruct((B, E), jnp.float32),
            jax.ShapeDtypeStruct((N,), jnp.float32),
        ],
        scratch_types=[
            pltpu.VMEM((PER_W,), jnp.int32),
            pltpu.VMEM((PER_W,), jnp.int32),
            pltpu.VMEM((CH, E), jnp.float32),
            pltpu.VMEM((CH, E), jnp.float32),
            pltpu.VMEM((PER_W, W1W), jnp.float32),
            pltpu.VMEM((PER_W,), jnp.float32),
            pltpu.VMEM((SPW, E), jnp.float32),
            pltpu.VMEM((SPW, E), jnp.float32),
            pltpu.SemaphoreType.DMA,
            pltpu.SemaphoreType.DMA,
            pltpu.SemaphoreType.DMA,
        ],
    )
    def k(vt, wt, idxh, semb_h, sqs_h, w1o_h,
          idx_v, idxhi_v, vbuf0, vbuf1, w1rows, w1sel_v, semb_all, sq_all,
          semv0, semv1, semw):
        wid = lax.axis_index("s") * NC + lax.axis_index("c")
        lbase = wid * PER_W
        sbase = wid * SPW
        pltpu.sync_copy(idxh.at[pl.ds(lbase, PER_W)], idx_v)

        @pl.loop(0, PER_W // 16)
        def _(g):
            s = pl.ds(g * 16, 16)
            idxhi_v[s] = lax.shift_right_logical(idx_v[s], 4)

        def fire_v(ci, buf, sem):
            pltpu.async_copy(vt.at[idx_v.at[pl.ds(ci * CH, CH)]], buf, sem)

        def wait_v(buf, sem):
            pltpu.make_async_copy(vt.at[pl.ds(0, CH)], buf, sem).wait()

        fire_v(0, vbuf0, semv0)

        @pl.loop(0, NWCH)
        def _(kk):
            off = kk * WCH
            pltpu.async_copy(wt.at[idxhi_v.at[pl.ds(off, WCH)]],
                             w1rows.at[pl.ds(off, WCH)], semw)

        def compute(buf, ci):
            for s in range(S_PER_CH):
                def fb(f, a, _s=s):
                    row = _s * F + f
                    v0 = buf[row, pl.ds(0, 16)]
                    v1 = buf[row, pl.ds(16, 16)]
                    v2 = buf[row, pl.ds(32, 16)]
                    v3 = buf[row, pl.ds(48, 16)]
                    return (a[0] + v0, a[1] + v1, a[2] + v2, a[3] + v3,
                            a[4] + v0 * v0, a[5] + v1 * v1,
                            a[6] + v2 * v2, a[7] + v3 * v3)
                z = jnp.zeros((16,), jnp.float32)
                acc = lax.fori_loop(0, F, fb, (z, z, z, z, z, z, z, z))
                samp = ci * S_PER_CH + s
                for c in range(4):
                    semb_all[samp, pl.ds(16 * c, 16)] = acc[c]
                    sq_all[samp, pl.ds(16 * c, 16)] = acc[4 + c]

        @pl.loop(0, NCH, step=2)
        def _(ci):
            fire_v(ci + 1, vbuf1, semv1)
            wait_v(vbuf0, semv0)
            compute(vbuf0, ci)

            @pl.when(ci + 2 < NCH)
            def _():
                fire_v(ci + 2, vbuf0, semv0)

            wait_v(vbuf1, semv1)
            compute(vbuf1, ci + 1)

        pltpu.make_async_copy(wt.at[pl.ds(0, PER_W)], w1rows, semw).wait()
        iota16 = lax.iota(jnp.int32, 16)

        @pl.loop(0, PER_W // 16)
        def _(g):
            s = pl.ds(g * 16, 16)
            lo = idx_v[s] & 15
            w1sel_v[s] = plsc.load_gather(w1rows, [iota16 + g * 16, lo])

        pltpu.sync_copy(semb_all, semb_h.at[pl.ds(sbase, SPW)])
        pltpu.sync_copy(sq_all, sqs_h.at[pl.ds(sbase, SPW)])
        pltpu.sync_copy(w1sel_v, w1o_h.at[pl.ds(lbase, PER_W)])

    return k(v_table, w1r16, idx_flat)


def _tc_body(semb_ref, sqs_ref, w1_ref, w0_ref, wl1_ref, bl1_ref, wl2_ref,
             bl2_ref, wl3_ref, bl3_ref, out_ref):
    semb = semb_ref[...]
    pp = (jnp.sum(semb * semb, axis=1, keepdims=True)
          + jnp.sum(sqs_ref[...], axis=1, keepdims=True)) * (1.0 / E)
    lin = jnp.sum(w1_ref[...], axis=1, keepdims=True) + w0_ref[...]
    memb = semb * (1.0 / F)
    h = jnp.dot(memb, wl1_ref[...], preferred_element_type=jnp.float32) + bl1_ref[...]
    h = jnp.where(h >= 0, h, 0.01 * h)
    h = jnp.dot(h, wl2_ref[...], preferred_element_type=jnp.float32) + bl2_ref[...]
    h = jnp.where(h >= 0, h, 0.01 * h)
    deep = jnp.dot(h, wl3_ref[...], preferred_element_type=jnp.float32) + bl3_ref[...]
    out_ref[...] = lin + 0.5 * pp + deep


def _tc_forward(semb, sqs, w1r, w0, wl1, bl1, wl2, bl2, wl3, bl3):
    return pl.pallas_call(
        _tc_body,
        out_shape=jax.ShapeDtypeStruct((B, 1), jnp.float32),
    )(semb, sqs, w1r, w0, wl1, bl1, wl2, bl2, wl3, bl3)


def kernel(x, W0, W1_table, V_table, W_l1, b_l1, W_l2, b_l2, W_l3, b_l3):
    idx_flat = x.reshape(-1).astype(jnp.int32)
    w1r16 = W1_table.reshape(VOCAB // W1W, W1W)
    semb, sqs, w1sel = _sc_fused(V_table, w1r16, idx_flat)
    return _tc_forward(
        semb, sqs, w1sel.reshape(B, F),
        W0.reshape(1, 1),
        W_l1, b_l1.reshape(1, 256),
        W_l2, b_l2.reshape(1, 128),
        W_l3, b_l3.reshape(1, 1),
    )


# speedup vs baseline: 2.4090x; 1.0854x over previous
import functools

import jax
import jax.numpy as jnp
from jax import lax
from jax.experimental import pallas as pl
from jax.experimental.pallas import tpu as pltpu
from jax.experimental.pallas import tpu_sc as plsc

B = 4096
F = 26
E = 64
N = B * F
VOCAB = 100000
NC, NS = 2, 16
NW = NC * NS
PER_W = N // NW
SPW = B // NW
S_PER_CH = 4
CH = S_PER_CH * F
NCH = PER_W // CH
W1W = 16
WCH = 128
NWCH = PER_W // WCH
CW = 128


def _sc_fused(v_table, w1r16, idx_flat):
    mesh = plsc.VectorSubcoreMesh(core_axis_name="c", subcore_axis_name="s")

    @functools.partial(
        pl.kernel,
        mesh=mesh,
        compiler_params=pltpu.CompilerParams(use_tc_tiling_on_sc=False,
                                             needs_layout_passes=False),
        out_type=jax.ShapeDtypeStruct((B, CW), jnp.float32),
        scratch_types=[
            pltpu.VMEM((PER_W,), jnp.int32),
            pltpu.VMEM((PER_W,), jnp.int32),
            pltpu.VMEM((CH, E), jnp.float32),
            pltpu.VMEM((CH, E), jnp.float32),
            pltpu.VMEM((PER_W, W1W), jnp.float32),
            pltpu.VMEM((SPW, CW), jnp.float32),
            pltpu.SemaphoreType.DMA,
            pltpu.SemaphoreType.DMA,
            pltpu.SemaphoreType.DMA,
        ],
    )
    def k(vt, wt, idxh, comb_h,
          idx_v, idxhi_v, vbuf0, vbuf1, w1rows, comb_all,
          semv0, semv1, semw):
        wid = lax.axis_index("s") * NC + lax.axis_index("c")
        lbase = wid * PER_W
        sbase = wid * SPW
        pltpu.sync_copy(idxh.at[pl.ds(lbase, PER_W)], idx_v)

        iota16 = lax.iota(jnp.int32, 16)

        @pl.loop(0, PER_W // 16)
        def _(g):
            s = pl.ds(g * 16, 16)
            idxhi_v[s] = lax.shift_right_logical(idx_v[s], 4)

        def fire_v(ci, buf, sem):
            pltpu.async_copy(vt.at[idx_v.at[pl.ds(ci * CH, CH)]], buf, sem)

        def wait_v(buf, sem):
            pltpu.make_async_copy(vt.at[pl.ds(0, CH)], buf, sem).wait()

        fire_v(0, vbuf0, semv0)

        @pl.loop(0, NWCH)
        def _(kk):
            off = kk * WCH
            pltpu.async_copy(wt.at[idxhi_v.at[pl.ds(off, WCH)]],
                             w1rows.at[pl.ds(off, WCH)], semw)

        def compute(buf, ci):
            for s in range(S_PER_CH):
                def fb(f, a, _s=s):
                    row = _s * F + f
                    v0 = buf[row, pl.ds(0, 16)]
                    v1 = buf[row, pl.ds(16, 16)]
                    v2 = buf[row, pl.ds(32, 16)]
                    v3 = buf[row, pl.ds(48, 16)]
                    return (a[0] + v0, a[1] + v1, a[2] + v2, a[3] + v3,
                            a[4] + v0 * v0, a[5] + v1 * v1,
                            a[6] + v2 * v2, a[7] + v3 * v3)
                z = jnp.zeros((16,), jnp.float32)
                acc = lax.fori_loop(0, F, fb, (z, z, z, z, z, z, z, z))
                samp = ci * S_PER_CH + s
                for c in range(4):
                    comb_all[samp, pl.ds(16 * c, 16)] = acc[c]
                sq = jnp.sum((acc[4] + acc[5]) + (acc[6] + acc[7]))
                sqv = jnp.where(iota16 == 0, sq, 0.0)
                comb_all[samp, pl.ds(E, 16)] = sqv

        @pl.loop(0, NCH, step=2)
        def _(ci):
            fire_v(ci + 1, vbuf1, semv1)
            wait_v(vbuf0, semv0)
            compute(vbuf0, ci)

            @pl.when(ci + 2 < NCH)
            def _():
                fire_v(ci + 2, vbuf0, semv0)

            wait_v(vbuf1, semv1)
            compute(vbuf1, ci + 1)

        pltpu.make_async_copy(wt.at[pl.ds(0, PER_W)], w1rows, semw).wait()

        @pl.loop(0, SPW // 16)
        def _(g):
            svec = iota16 + g * 16

            def wb(f, a):
                jvec = svec * F + f
                ivec = plsc.load_gather(idx_v, [jvec])
                vals = plsc.load_gather(w1rows, [jvec, ivec & 15])
                return a + vals

            acc = lax.fori_loop(0, F, wb, jnp.zeros((16,), jnp.float32))
            plsc.store_scatter(comb_all, [svec, jnp.full((16,), E + 1, jnp.int32)], acc)

        pltpu.sync_copy(comb_all, comb_h.at[pl.ds(sbase, SPW)])

    return k(v_table, w1r16, idx_flat)


def _tc_body(comb_ref, w0_ref, wl1_ref, bl1_ref, wl2_ref, bl2_ref,
             wl3_ref, bl3_ref, out_ref):
    comb = comb_ref[...]
    semb = comb[:, 0:E]
    pp = (jnp.sum(semb * semb, axis=1, keepdims=True)
          + comb[:, E:E + 1]) * (1.0 / E)
    lin = comb[:, E + 1:E + 2] + w0_ref[...]
    memb = semb * (1.0 / F)
    h = jnp.dot(memb, wl1_ref[...], preferred_element_type=jnp.float32) + bl1_ref[...]
    h = jnp.where(h >= 0, h, 0.01 * h)
    h = jnp.dot(h, wl2_ref[...], preferred_element_type=jnp.float32) + bl2_ref[...]
    h = jnp.where(h >= 0, h, 0.01 * h)
    deep = jnp.dot(h, wl3_ref[...], preferred_element_type=jnp.float32) + bl3_ref[...]
    out_ref[...] = lin + 0.5 * pp + deep


def _tc_forward(comb, w0, wl1, bl1, wl2, bl2, wl3, bl3):
    return pl.pallas_call(
        _tc_body,
        out_shape=jax.ShapeDtypeStruct((B, 1), jnp.float32),
    )(comb, w0, wl1, bl1, wl2, bl2, wl3, bl3)


def kernel(x, W0, W1_table, V_table, W_l1, b_l1, W_l2, b_l2, W_l3, b_l3):
    idx_flat = x.reshape(-1).astype(jnp.int32)
    w1r16 = W1_table.reshape(VOCAB // W1W, W1W)
    comb = _sc_fused(V_table, w1r16, idx_flat)
    return _tc_forward(
        comb,
        W0.reshape(1, 1),
        W_l1, b_l1.reshape(1, 256),
        W_l2, b_l2.reshape(1, 128),
        W_l3, b_l3.reshape(1, 1),
    )


# speedup vs baseline: 2.4119x; 1.0012x over previous
import functools

import jax
import jax.numpy as jnp
from jax import lax
from jax.experimental import pallas as pl
from jax.experimental.pallas import tpu as pltpu
from jax.experimental.pallas import tpu_sc as plsc

B = 4096
F = 26
E = 64
EP = 128
N = B * F
VOCAB = 100000
NC, NS = 2, 16
NW = NC * NS
PER_W = N // NW
SPW = B // NW
S_PER_CH = 4
CH = S_PER_CH * F
NCH = PER_W // CH
W1W = 16
WCH = 128
NWCH = PER_W // WCH
CW = 128


def _sc_v(v_pad, idx_flat):
    mesh = plsc.VectorSubcoreMesh(core_axis_name="c", subcore_axis_name="s")

    @functools.partial(
        pl.kernel,
        mesh=mesh,
        compiler_params=pltpu.CompilerParams(use_tc_tiling_on_sc=True,
                                             needs_layout_passes=False),
        out_type=jax.ShapeDtypeStruct((B, CW), jnp.float32),
        scratch_types=[
            pltpu.VMEM((PER_W,), jnp.int32),
            pltpu.VMEM((CH, EP), jnp.float32),
            pltpu.VMEM((CH, EP), jnp.float32),
            pltpu.VMEM((SPW, CW), jnp.float32),
            pltpu.SemaphoreType.DMA,
            pltpu.SemaphoreType.DMA,
        ],
    )
    def k(vt, idxh, comb_h, idx_v, vbuf0, vbuf1, comb_all, semv0, semv1):
        wid = lax.axis_index("s") * NC + lax.axis_index("c")
        lbase = wid * PER_W
        sbase = wid * SPW
        pltpu.sync_copy(idxh.at[pl.ds(lbase, PER_W)], idx_v)

        iota16 = lax.iota(jnp.int32, 16)

        def fire_v(ci, buf, sem):
            pltpu.async_copy(vt.at[idx_v.at[pl.ds(ci * CH, CH)]], buf, sem)

        def wait_v(buf, sem):
            pltpu.make_async_copy(vt.at[pl.ds(0, CH)], buf, sem).wait()

        fire_v(0, vbuf0, semv0)

        def compute(buf, ci):
            for s in range(S_PER_CH):
                def fb(f, a, _s=s):
                    row = _s * F + f
                    v0 = buf[row, pl.ds(0, 16)]
                    v1 = buf[row, pl.ds(16, 16)]
                    v2 = buf[row, pl.ds(32, 16)]
                    v3 = buf[row, pl.ds(48, 16)]
                    return (a[0] + v0, a[1] + v1, a[2] + v2, a[3] + v3,
                            a[4] + v0 * v0, a[5] + v1 * v1,
                            a[6] + v2 * v2, a[7] + v3 * v3)
                z = jnp.zeros((16,), jnp.float32)
                acc = lax.fori_loop(0, F, fb, (z, z, z, z, z, z, z, z))
                samp = ci * S_PER_CH + s
                for c in range(4):
                    comb_all[samp, pl.ds(16 * c, 16)] = acc[c]
                sq = jnp.sum((acc[4] + acc[5]) + (acc[6] + acc[7]))
                sqv = jnp.where(iota16 == 0, sq, 0.0)
                comb_all[samp, pl.ds(E, 16)] = sqv

        @pl.loop(0, NCH, step=2)
        def _(ci):
            fire_v(ci + 1, vbuf1, semv1)
            wait_v(vbuf0, semv0)
            compute(vbuf0, ci)

            @pl.when(ci + 2 < NCH)
            def _():
                fire_v(ci + 2, vbuf0, semv0)

            wait_v(vbuf1, semv1)
            compute(vbuf1, ci + 1)

        pltpu.sync_copy(comb_all, comb_h.at[pl.ds(sbase, SPW)])

    return k(v_pad, idx_flat)


def _sc_w1(w1r16, idx_flat):
    mesh = plsc.VectorSubcoreMesh(core_axis_name="c", subcore_axis_name="s")

    @functools.partial(
        pl.kernel,
        mesh=mesh,
        compiler_params=pltpu.CompilerParams(use_tc_tiling_on_sc=False,
                                             needs_layout_passes=False),
        out_type=jax.ShapeDtypeStruct((B,), jnp.float32),
        scratch_types=[
            pltpu.VMEM((PER_W,), jnp.int32),
            pltpu.VMEM((PER_W,), jnp.int32),
            pltpu.VMEM((PER_W, W1W), jnp.float32),
            pltpu.VMEM((SPW,), jnp.float32),
            pltpu.SemaphoreType.DMA,
        ],
    )
    def k(wt, idxh, w1o_h, idx_v, idxhi_v, w1rows, w1s_v, semw):
        wid = lax.axis_index("s") * NC + lax.axis_index("c")
        lbase = wid * PER_W
        sbase = wid * SPW
        pltpu.sync_copy(idxh.at[pl.ds(lbase, PER_W)], idx_v)

        iota16 = lax.iota(jnp.int32, 16)

        @pl.loop(0, PER_W // 16)
        def _(g):
            s = pl.ds(g * 16, 16)
            idxhi_v[s] = lax.shift_right_logical(idx_v[s], 4)

        @pl.loop(0, NWCH)
        def _(kk):
            off = kk * WCH
            pltpu.async_copy(wt.at[idxhi_v.at[pl.ds(off, WCH)]],
                             w1rows.at[pl.ds(off, WCH)], semw)

        pltpu.make_async_copy(wt.at[pl.ds(0, PER_W)], w1rows, semw).wait()

        @pl.loop(0, SPW // 16)
        def _(g):
            svec = iota16 + g * 16

            def wb(f, a):
                jvec = svec * F + f
                ivec = plsc.load_gather(idx_v, [jvec])
                vals = plsc.load_gather(w1rows, [jvec, ivec & 15])
                return a + vals

            acc = lax.fori_loop(0, F, wb, jnp.zeros((16,), jnp.float32))
            w1s_v[pl.ds(g * 16, 16)] = acc

        pltpu.sync_copy(w1s_v, w1o_h.at[pl.ds(sbase, SPW)])

    return k(w1r16, idx_flat)


def _tc_body(comb_ref, w1_ref, w0_ref, wl1_ref, bl1_ref, wl2_ref, bl2_ref,
             wl3_ref, bl3_ref, out_ref):
    comb = comb_ref[...]
    semb = comb[:, 0:E]
    pp = (jnp.sum(semb * semb, axis=1, keepdims=True)
          + comb[:, E:E + 1]) * (1.0 / E)
    lin = w1_ref[...] + w0_ref[...]
    memb = semb * (1.0 / F)
    h = jnp.dot(memb, wl1_ref[...], preferred_element_type=jnp.float32) + bl1_ref[...]
    h = jnp.where(h >= 0, h, 0.01 * h)
    h = jnp.dot(h, wl2_ref[...], preferred_element_type=jnp.float32) + bl2_ref[...]
    h = jnp.where(h >= 0, h, 0.01 * h)
    deep = jnp.dot(h, wl3_ref[...], preferred_element_type=jnp.float32) + bl3_ref[...]
    out_ref[...] = lin + 0.5 * pp + deep


def _tc_forward(comb, w1s, w0, wl1, bl1, wl2, bl2, wl3, bl3):
    return pl.pallas_call(
        _tc_body,
        out_shape=jax.ShapeDtypeStruct((B, 1), jnp.float32),
    )(comb, w1s, w0, wl1, bl1, wl2, bl2, wl3, bl3)


def kernel(x, W0, W1_table, V_table, W_l1, b_l1, W_l2, b_l2, W_l3, b_l3):
    idx_flat = x.reshape(-1).astype(jnp.int32)
    w1r16 = W1_table.reshape(VOCAB // W1W, W1W)
    v_pad = jnp.pad(V_table, ((0, 0), (0, EP - E)))
    comb = _sc_v(v_pad, idx_flat)
    w1s = _sc_w1(w1r16, idx_flat)
    return _tc_forward(
        comb, w1s.reshape(B, 1),
        W0.reshape(1, 1),
        W_l1, b_l1.reshape(1, 256),
        W_l2, b_l2.reshape(1, 128),
        W_l3, b_l3.reshape(1, 1),
    )


# speedup vs baseline: 2.4565x; 1.0185x over previous
import functools

import jax
import jax.numpy as jnp
from jax import lax
from jax.experimental import pallas as pl
from jax.experimental.pallas import tpu as pltpu
from jax.experimental.pallas import tpu_sc as plsc

B = 4096
F = 26
E = 64
EP = 128
N = B * F
VOCAB = 100000
NC, NS = 2, 16
NW = NC * NS
PER_W = N // NW
SPW = B // NW
S_PER_CH = 4
CH = S_PER_CH * F
NCH = PER_W // CH
W1W = 16
WCH = 128
NWCH = PER_W // WCH
CW = 128


def _sc_v(v_table, idx_flat):
    mesh = plsc.VectorSubcoreMesh(core_axis_name="c", subcore_axis_name="s")

    @functools.partial(
        pl.kernel,
        mesh=mesh,
        compiler_params=pltpu.CompilerParams(use_tc_tiling_on_sc=False,
                                             needs_layout_passes=False),
        out_type=jax.ShapeDtypeStruct((B, CW), jnp.float32),
        scratch_types=[
            pltpu.VMEM((PER_W,), jnp.int32),
            pltpu.VMEM((CH, E), jnp.float32),
            pltpu.VMEM((CH, E), jnp.float32),
            pltpu.VMEM((SPW, CW), jnp.float32),
            pltpu.SemaphoreType.DMA,
            pltpu.SemaphoreType.DMA,
        ],
    )
    def k(vt, idxh, comb_h, idx_v, vbuf0, vbuf1, comb_all, semv0, semv1):
        wid = lax.axis_index("s") * NC + lax.axis_index("c")
        lbase = wid * PER_W
        sbase = wid * SPW
        pltpu.sync_copy(idxh.at[pl.ds(lbase, PER_W)], idx_v)

        iota16 = lax.iota(jnp.int32, 16)

        def fire_v(ci, buf, sem):
            pltpu.async_copy(vt.at[idx_v.at[pl.ds(ci * CH, CH)]], buf, sem)

        def wait_v(buf, sem):
            pltpu.make_async_copy(vt.at[pl.ds(0, CH)], buf, sem).wait()

        fire_v(0, vbuf0, semv0)

        def compute(buf, ci):
            for s in range(S_PER_CH):
                def fb(f, a, _s=s):
                    row = _s * F + f
                    v0 = buf[row, pl.ds(0, 16)]
                    v1 = buf[row, pl.ds(16, 16)]
                    v2 = buf[row, pl.ds(32, 16)]
                    v3 = buf[row, pl.ds(48, 16)]
                    return (a[0] + v0, a[1] + v1, a[2] + v2, a[3] + v3,
                            a[4] + v0 * v0, a[5] + v1 * v1,
                            a[6] + v2 * v2, a[7] + v3 * v3)
                z = jnp.zeros((16,), jnp.float32)
                acc = lax.fori_loop(0, F, fb, (z, z, z, z, z, z, z, z))
                samp = ci * S_PER_CH + s
                for c in range(4):
                    comb_all[samp, pl.ds(16 * c, 16)] = acc[c]
                sq = jnp.sum((acc[4] + acc[5]) + (acc[6] + acc[7]))
                sqv = jnp.where(iota16 == 0, sq, 0.0)
                comb_all[samp, pl.ds(E, 16)] = sqv

        @pl.loop(0, NCH, step=2)
        def _(ci):
            fire_v(ci + 1, vbuf1, semv1)
            wait_v(vbuf0, semv0)
            compute(vbuf0, ci)

            @pl.when(ci + 2 < NCH)
            def _():
                fire_v(ci + 2, vbuf0, semv0)

            wait_v(vbuf1, semv1)
            compute(vbuf1, ci + 1)

        pltpu.sync_copy(comb_all, comb_h.at[pl.ds(sbase, SPW)])

    return k(v_table, idx_flat)


def _sc_w1(w1r16, idx_flat):
    mesh = plsc.VectorSubcoreMesh(core_axis_name="c", subcore_axis_name="s")

    @functools.partial(
        pl.kernel,
        mesh=mesh,
        compiler_params=pltpu.CompilerParams(use_tc_tiling_on_sc=False,
                                             needs_layout_passes=False),
        out_type=jax.ShapeDtypeStruct((B,), jnp.float32),
        scratch_types=[
            pltpu.VMEM((PER_W,), jnp.int32),
            pltpu.VMEM((PER_W,), jnp.int32),
            pltpu.VMEM((PER_W, W1W), jnp.float32),
            pltpu.VMEM((SPW,), jnp.float32),
            pltpu.SemaphoreType.DMA,
        ],
    )
    def k(wt, idxh, w1o_h, idx_v, idxhi_v, w1rows, w1s_v, semw):
        wid = lax.axis_index("s") * NC + lax.axis_index("c")
        lbase = wid * PER_W
        sbase = wid * SPW
        pltpu.sync_copy(idxh.at[pl.ds(lbase, PER_W)], idx_v)

        iota16 = lax.iota(jnp.int32, 16)

        @pl.loop(0, PER_W // 16)
        def _(g):
            s = pl.ds(g * 16, 16)
            idxhi_v[s] = lax.shift_right_logical(idx_v[s], 4)

        @pl.loop(0, NWCH)
        def _(kk):
            off = kk * WCH
            pltpu.async_copy(wt.at[idxhi_v.at[pl.ds(off, WCH)]],
                             w1rows.at[pl.ds(off, WCH)], semw)

        pltpu.make_async_copy(wt.at[pl.ds(0, PER_W)], w1rows, semw).wait()

        @pl.loop(0, SPW // 16)
        def _(g):
            svec = iota16 + g * 16

            def wb(f, a):
                jvec = svec * F + f
                ivec = plsc.load_gather(idx_v, [jvec])
                vals = plsc.load_gather(w1rows, [jvec, ivec & 15])
                return a + vals

            acc = lax.fori_loop(0, F, wb, jnp.zeros((16,), jnp.float32))
            w1s_v[pl.ds(g * 16, 16)] = acc

        pltpu.sync_copy(w1s_v, w1o_h.at[pl.ds(sbase, SPW)])

    return k(w1r16, idx_flat)


def _tc_body(comb_ref, w1_ref, w0_ref, wl1_ref, bl1_ref, wl2_ref, bl2_ref,
             wl3_ref, bl3_ref, out_ref):
    comb = comb_ref[...]
    semb = comb[:, 0:E]
    pp = (jnp.sum(semb * semb, axis=1, keepdims=True)
          + comb[:, E:E + 1]) * (1.0 / E)
    lin = w1_ref[...] + w0_ref[...]
    memb = semb * (1.0 / F)
    h = jnp.dot(memb, wl1_ref[...], preferred_element_type=jnp.float32) + bl1_ref[...]
    h = jnp.where(h >= 0, h, 0.01 * h)
    h = jnp.dot(h, wl2_ref[...], preferred_element_type=jnp.float32) + bl2_ref[...]
    h = jnp.where(h >= 0, h, 0.01 * h)
    deep = jnp.dot(h, wl3_ref[...], preferred_element_type=jnp.float32) + bl3_ref[...]
    out_ref[...] = lin + 0.5 * pp + deep


def _tc_forward(comb, w1s, w0, wl1, bl1, wl2, bl2, wl3, bl3):
    return pl.pallas_call(
        _tc_body,
        out_shape=jax.ShapeDtypeStruct((B, 1), jnp.float32),
    )(comb, w1s, w0, wl1, bl1, wl2, bl2, wl3, bl3)


def kernel(x, W0, W1_table, V_table, W_l1, b_l1, W_l2, b_l2, W_l3, b_l3):
    idx_flat = x.reshape(-1).astype(jnp.int32)
    w1r16 = W1_table.reshape(VOCAB // W1W, W1W)
    comb = _sc_v(V_table, idx_flat)
    w1s = _sc_w1(w1r16, idx_flat)
    return _tc_forward(
        comb, w1s.reshape(B, 1),
        W0.reshape(1, 1),
        W_l1, b_l1.reshape(1, 256),
        W_l2, b_l2.reshape(1, 128),
        W_l3, b_l3.reshape(1, 1),
    )
